# baseline (speedup 1.0000x reference)
import functools

import jax
import jax.numpy as jnp
from jax import lax
from jax.experimental import pallas as pl
from jax.experimental.pallas import tpu as pltpu
from jax.experimental.pallas import tpu_sc as plsc

_OH = 7
_OW = 7
_TROWS = 1032


def _build_body(feat_ref, t_ref):
    lk = pl.program_id(1)
    l = lk // 3
    k = lk % 3
    t0 = feat_ref[0]
    x1 = jnp.maximum(t0, jnp.concatenate([t0[1:], t0[-1:]], axis=0))
    x2 = jnp.maximum(x1, jnp.concatenate([x1[2:], x1[-2:]], axis=0))
    tx = jnp.where(k == 0, t0, jnp.where(k == 1, x1, x2))
    y1 = jnp.maximum(tx, jnp.concatenate([tx[32:], tx[-32:]], axis=0))
    y2 = jnp.maximum(y1, jnp.concatenate([y1[64:], y1[-64:]], axis=0))
    ty = jnp.where(l == 0, tx, jnp.where(l == 1, y1, y2))
    t_ref[0, 0, : t0.shape[0]] = ty
    t_ref[0, 0, t0.shape[0] :] = jnp.zeros(
        (_TROWS - t0.shape[0], t0.shape[1]), t0.dtype
    )


def _sc_roi_kernel(n_workers, rois_per_worker, c, out_rows):
    mesh = plsc.VectorSubcoreMesh(core_axis_name="c", subcore_axis_name="s")

    @functools.partial(
        pl.kernel,
        mesh=mesh,
        out_type=jax.ShapeDtypeStruct((out_rows, 56, c), jnp.float32),
        scratch_types=[
            pltpu.VMEM((4 * 56,), jnp.int32),
            pltpu.VMEM((56, c), jnp.float32),
            pltpu.VMEM((56, c), jnp.float32),
            pltpu.SemaphoreType.DMA,
            pltpu.SemaphoreType.DMA,
        ],
    )
    def sck(tbl_hbm, idx_hbm, out_hbm, idx_v, acc_v, buf_v, sem_a, sem_b):
        wid = lax.axis_index("s") * 2 + lax.axis_index("c")

        def one_roi(r, carry):
            roi = wid * rois_per_worker + r
            pltpu.sync_copy(idx_hbm.at[pl.ds(roi * 4 * 56, 4 * 56)], idx_v)
            pltpu.async_copy(
                tbl_hbm.at[idx_v.at[pl.ds(0, 56)]], acc_v, sem_a
            ).wait()

            def merge_corner(corner, carry2):
                pltpu.async_copy(
                    tbl_hbm.at[idx_v.at[pl.ds(corner * 56, 56)]], buf_v, sem_b
                ).wait()

                def merge_row(row, carry3):
                    def merge_chunk(q, carry4):
                        sl = pl.ds(q * 16, 16)
                        acc_v[row, sl] = jnp.maximum(
                            acc_v[row, sl], buf_v[row, sl]
                        )
                        return carry4

                    return lax.fori_loop(0, c // 16, merge_chunk, carry3)

                return lax.fori_loop(0, _OH * _OW, merge_row, carry2)

            lax.fori_loop(1, 4, merge_corner, 0)
            pltpu.sync_copy(acc_v, out_hbm.at[roi])
            return carry

        lax.fori_loop(0, rois_per_worker, one_roi, 0)

    return sck


def kernel(features, rois):
    b, c, h, w = features.shape
    n = rois.shape[1]

    x1 = jnp.maximum(0, (rois[..., 0] * w).astype(jnp.int32))
    y1 = jnp.maximum(0, (rois[..., 1] * h).astype(jnp.int32))
    x2 = jnp.minimum(w - 1, (rois[..., 2] * w).astype(jnp.int32))
    y2 = jnp.minimum(h - 1, (rois[..., 3] * h).astype(jnp.int32))
    valid = (x2 >= x1) & (y2 >= y1)
    rw = x2 - x1 + 1
    rh = y2 - y1 + 1
    jj = jnp.arange(_OW)
    ii = jnp.arange(_OH)
    xs = x1[..., None] + (jj * rw[..., None]) // _OW
    xe = x1[..., None] + -((-(jj + 1) * rw[..., None]) // _OW)
    ys = y1[..., None] + (ii * rh[..., None]) // _OH
    ye = y1[..., None] + -((-(ii + 1) * rh[..., None]) // _OH)
    lenx = jnp.maximum(xe - xs, 1)
    kx = (lenx >= 2).astype(jnp.int32) + (lenx >= 4).astype(jnp.int32)
    xb = xe - jnp.left_shift(1, kx)
    leny = jnp.maximum(ye - ys, 1)
    ky = (leny >= 2).astype(jnp.int32) + (leny >= 4).astype(jnp.int32)
    yb = ye - jnp.left_shift(1, ky)

    bnum = jnp.arange(b, dtype=jnp.int32)[:, None, None, None]
    combo = (ky[..., :, None] * 3 + kx[..., None, :])
    base = (bnum * 9 + combo) * _TROWS
    ya_r = ys[..., :, None] * w
    yb_r = yb[..., :, None] * w
    xs_c = xs[..., None, :]
    xb_c = xb[..., None, :]
    corners = jnp.stack(
        [
            base + ya_r + xs_c,
            base + ya_r + xb_c,
            base + yb_r + xs_c,
            base + yb_r + xb_c,
        ],
        axis=2,
    )
    zid = h * w
    corners = jnp.where(valid[..., None, None, None], corners, zid)
    corners = corners.reshape(b, n, 4, _OH * _OW)
    pad = jnp.full((b, n, 4, 56 - _OH * _OW), zid, jnp.int32)
    idx = jnp.concatenate([corners, pad], axis=-1).reshape(-1)

    feat2d = features.transpose(0, 2, 3, 1).reshape(b, h * w, c)

    tables = pl.pallas_call(
        _build_body,
        grid=(b, 9),
        in_specs=[pl.BlockSpec((1, h * w, c), lambda pb, lk: (pb, 0, 0))],
        out_specs=pl.BlockSpec(
            (1, 1, _TROWS, c), lambda pb, lk: (pb, lk, 0, 0)
        ),
        out_shape=jax.ShapeDtypeStruct((b, 9, _TROWS, c), jnp.float32),
    )(feat2d)
    tbl = tables.reshape(b * 9 * _TROWS, c)

    n_workers = 32
    rois_per_worker = (b * n) // n_workers
    out = _sc_roi_kernel(n_workers, rois_per_worker, c, b * n)(tbl, idx)

    out = out[:, : _OH * _OW, :].reshape(b, n, _OH * _OW, c)
    return out.transpose(0, 1, 3, 2).reshape(b, n, c, _OH, _OW)


# speedup vs baseline: 1.0277x; 1.0277x over previous
import functools

import jax
import jax.numpy as jnp
from jax import lax
from jax.experimental import pallas as pl
from jax.experimental.pallas import tpu as pltpu
from jax.experimental.pallas import tpu_sc as plsc

_OH = 7
_OW = 7
_NCELL = _OH * _OW
_TROWS = 1032
_N_SC = 32


def _build_body(feat_ref, t_ref):
    lk = pl.program_id(1)
    l = lk // 3
    k = lk % 3
    t0 = feat_ref[0]
    x1 = jnp.maximum(t0, jnp.concatenate([t0[1:], t0[-1:]], axis=0))
    x2 = jnp.maximum(x1, jnp.concatenate([x1[2:], x1[-2:]], axis=0))
    tx = jnp.where(k == 0, t0, jnp.where(k == 1, x1, x2))
    y1 = jnp.maximum(tx, jnp.concatenate([tx[32:], tx[-32:]], axis=0))
    y2 = jnp.maximum(y1, jnp.concatenate([y1[64:], y1[-64:]], axis=0))
    ty = jnp.where(l == 0, tx, jnp.where(l == 1, y1, y2))
    t_ref[0, 0, : t0.shape[0]] = ty
    t_ref[0, 0, t0.shape[0] :] = jnp.zeros(
        (_TROWS - t0.shape[0], t0.shape[1]), t0.dtype
    )


def _sc_roi_kernel(rois_per_worker, c, out_rows):
    mesh = plsc.VectorSubcoreMesh(core_axis_name="c", subcore_axis_name="s")

    @functools.partial(
        pl.kernel,
        mesh=mesh,
        out_type=jax.ShapeDtypeStruct((out_rows, 56, c), jnp.float32),
        scratch_types=[
            pltpu.VMEM((4 * 56,), jnp.int32),
            pltpu.VMEM((56, c), jnp.float32),
            pltpu.VMEM((56, c), jnp.float32),
            pltpu.SemaphoreType.DMA,
            pltpu.SemaphoreType.DMA,
        ],
    )
    def sck(tbl_hbm, idx_hbm, out_hbm, idx_v, acc_v, buf_v, sem_a, sem_b):
        wid = lax.axis_index("s") * 2 + lax.axis_index("c")

        def one_roi(r, carry):
            roi = wid * rois_per_worker + r
            pltpu.sync_copy(idx_hbm.at[pl.ds(roi * 4 * 56, 4 * 56)], idx_v)
            pltpu.async_copy(
                tbl_hbm.at[idx_v.at[pl.ds(0, 56)]], acc_v, sem_a
            ).wait()

            def merge_corner(corner, carry2):
                pltpu.async_copy(
                    tbl_hbm.at[idx_v.at[pl.ds(corner * 56, 56)]], buf_v, sem_b
                ).wait()

                def merge_row(row, carry3):
                    def merge_chunk(q, carry4):
                        sl = pl.ds(q * 16, 16)
                        acc_v[row, sl] = jnp.maximum(
                            acc_v[row, sl], buf_v[row, sl]
                        )
                        return carry4

                    return lax.fori_loop(
                        0, c // 16, merge_chunk, carry3, unroll=8
                    )

                return lax.fori_loop(0, _NCELL, merge_row, carry2)

            lax.fori_loop(1, 4, merge_corner, 0)
            pltpu.sync_copy(acc_v, out_hbm.at[roi])
            return carry

        lax.fori_loop(0, rois_per_worker, one_roi, 0)

    return sck


def _tc_roi_body(bounds_ref, feat_ref, out_ref, tx_ref):
    pb = pl.program_id(0)
    pn = pl.program_id(1)
    h = feat_ref.shape[2]

    @pl.when(pn == 0)
    def _build():
        t0 = feat_ref[0]
        t1 = jnp.maximum(t0, jnp.concatenate([t0[1:], t0[-1:]], axis=0))
        t2 = jnp.maximum(t1, jnp.concatenate([t1[2:], t1[-2:]], axis=0))
        tx_ref[0] = t0
        tx_ref[1] = t1
        tx_ref[2] = t2

    cms = []
    for jj in range(_OW):
        xs = bounds_ref[pb, pn, jj]
        xb = bounds_ref[pb, pn, _OW + jj]
        kx = bounds_ref[pb, pn, 2 * _OW + jj]
        cms.append(jnp.maximum(tx_ref[kx, xs], tx_ref[kx, xb]))
    cmall = jnp.concatenate(cms, axis=-1)

    neg = jnp.array(-jnp.inf, dtype=cmall.dtype)
    zero = jnp.array(0.0, dtype=cmall.dtype)
    ridx = jax.lax.broadcasted_iota(jnp.int32, (h, 1), 0)
    vflag = bounds_ref[pb, pn, 5 * _OW]
    for ii in range(_OH):
        ys = bounds_ref[pb, pn, 3 * _OW + ii]
        ye = bounds_ref[pb, pn, 4 * _OW + ii]
        rm = (ridx >= ys) & (ridx < ye)
        row = jnp.max(jnp.where(rm, cmall, neg), axis=0)
        out_ref[0, 0, ii, :] = jnp.where(vflag > 0, row, zero)


def kernel(features, rois):
    b, c, h, w = features.shape
    n = rois.shape[1]
    n_sc = _N_SC
    n_tc = n - n_sc

    x1 = jnp.maximum(0, (rois[..., 0] * w).astype(jnp.int32))
    y1 = jnp.maximum(0, (rois[..., 1] * h).astype(jnp.int32))
    x2 = jnp.minimum(w - 1, (rois[..., 2] * w).astype(jnp.int32))
    y2 = jnp.minimum(h - 1, (rois[..., 3] * h).astype(jnp.int32))
    valid = (x2 >= x1) & (y2 >= y1)
    rw = x2 - x1 + 1
    rh = y2 - y1 + 1
    jjj = jnp.arange(_OW)
    iii = jnp.arange(_OH)
    xs = x1[..., None] + (jjj * rw[..., None]) // _OW
    xe = x1[..., None] + -((-(jjj + 1) * rw[..., None]) // _OW)
    ys = y1[..., None] + (iii * rh[..., None]) // _OH
    ye = y1[..., None] + -((-(iii + 1) * rh[..., None]) // _OH)
    lenx = jnp.maximum(xe - xs, 1)
    kx = (lenx >= 2).astype(jnp.int32) + (lenx >= 4).astype(jnp.int32)
    xb = xe - jnp.left_shift(1, kx)
    leny = jnp.maximum(ye - ys, 1)
    ky = (leny >= 2).astype(jnp.int32) + (leny >= 4).astype(jnp.int32)
    yb = ye - jnp.left_shift(1, ky)

    bounds = jnp.concatenate(
        [
            jnp.clip(xs, 0, w - 1),
            jnp.clip(xb, 0, w - 1),
            kx,
            ys,
            ye,
            valid[..., None].astype(jnp.int32),
        ],
        axis=-1,
    )[:, :n_tc]

    feat_t = features.transpose(0, 3, 2, 1)

    out_tc = pl.pallas_call(
        _tc_roi_body,
        grid_spec=pltpu.PrefetchScalarGridSpec(
            num_scalar_prefetch=1,
            grid=(b, n_tc),
            in_specs=[
                pl.BlockSpec((1, w, h, c), lambda pb, pn, bnds: (pb, 0, 0, 0)),
            ],
            out_specs=pl.BlockSpec(
                (1, 1, _OH, _OW * c), lambda pb, pn, bnds: (pb, pn, 0, 0)
            ),
            scratch_shapes=[pltpu.VMEM((3, w, h, c), features.dtype)],
        ),
        out_shape=jax.ShapeDtypeStruct((b, n_tc, _OH, _OW * c), features.dtype),
    )(bounds, feat_t)

    bnum = jnp.arange(b, dtype=jnp.int32)[:, None, None, None]
    combo = ky[..., :, None] * 3 + kx[..., None, :]
    base = (bnum * 9 + combo) * _TROWS
    ya_r = ys[..., :, None] * w
    yb_r = yb[..., :, None] * w
    xs_c = xs[..., None, :]
    xb_c = xb[..., None, :]
    corners = jnp.stack(
        [
            base + ya_r + xs_c,
            base + ya_r + xb_c,
            base + yb_r + xs_c,
            base + yb_r + xb_c,
        ],
        axis=2,
    )
    zid = h * w
    corners = jnp.where(valid[..., None, None, None], corners, zid)
    corners = corners[:, n_tc:].reshape(b, n_sc, 4, _NCELL)
    pad = jnp.full((b, n_sc, 4, 56 - _NCELL), zid, jnp.int32)
    idx = jnp.concatenate([corners, pad], axis=-1).reshape(-1)

    feat2d = features.transpose(0, 2, 3, 1).reshape(b, h * w, c)
    tables = pl.pallas_call(
        _build_body,
        grid=(b, 9),
        in_specs=[pl.BlockSpec((1, h * w, c), lambda pb, lk: (pb, 0, 0))],
        out_specs=pl.BlockSpec(
            (1, 1, _TROWS, c), lambda pb, lk: (pb, lk, 0, 0)
        ),
        out_shape=jax.ShapeDtypeStruct((b, 9, _TROWS, c), jnp.float32),
    )(feat2d)
    tbl = tables.reshape(b * 9 * _TROWS, c)

    rois_per_worker = (b * n_sc) // 32
    out_sc = _sc_roi_kernel(rois_per_worker, c, b * n_sc)(tbl, idx)

    o_tc = (
        out_tc.reshape(b, n_tc, _OH, _OW, c).transpose(0, 1, 4, 2, 3)
    )
    o_sc = (
        out_sc[:, :_NCELL].reshape(b, n_sc, _NCELL, c)
        .transpose(0, 1, 3, 2)
        .reshape(b, n_sc, c, _OH, _OW)
    )
    return jnp.concatenate([o_tc, o_sc], axis=1)


# speedup vs baseline: 1.8873x; 1.8364x over previous
import jax
import jax.numpy as jnp
from jax.experimental import pallas as pl
from jax.experimental.pallas import tpu as pltpu

_OH = 7
_OW = 7


def _roi_body(bounds_ref, feat_ref, out_ref, tx_ref):
    pb = pl.program_id(0)
    pn = pl.program_id(1)
    h = feat_ref.shape[2]

    @pl.when(pn == 0)
    def _build():
        t0 = feat_ref[0]
        t1 = jnp.maximum(t0, jnp.concatenate([t0[1:], t0[-1:]], axis=0))
        t2 = jnp.maximum(t1, jnp.concatenate([t1[2:], t1[-2:]], axis=0))
        tx_ref[0] = t0
        tx_ref[1] = t1
        tx_ref[2] = t2

    cms = []
    for jj in range(_OW):
        xs = bounds_ref[pb, pn, jj]
        xb = bounds_ref[pb, pn, _OW + jj]
        kx = bounds_ref[pb, pn, 2 * _OW + jj]
        cms.append(jnp.maximum(tx_ref[kx, xs], tx_ref[kx, xb]))
    cmall = jnp.concatenate(cms, axis=-1)

    neg = jnp.array(-jnp.inf, dtype=cmall.dtype)
    zero = jnp.array(0.0, dtype=jnp.float32)
    ridx = jax.lax.broadcasted_iota(jnp.int32, (h, 1), 0)
    vflag = bounds_ref[pb, pn, 5 * _OW]
    for ii in range(_OH):
        ys = bounds_ref[pb, pn, 3 * _OW + ii]
        ye = bounds_ref[pb, pn, 4 * _OW + ii]
        rm = (ridx >= ys) & (ridx < ye)
        row = jnp.max(jnp.where(rm, cmall, neg), axis=0)
        row = row.astype(jnp.float32)
        out_ref[0, 0, ii, :] = jnp.where(vflag > 0, row, zero)


def kernel(features, rois):
    b, c, h, w = features.shape
    n = rois.shape[1]

    x1 = jnp.maximum(0, (rois[..., 0] * w).astype(jnp.int32))
    y1 = jnp.maximum(0, (rois[..., 1] * h).astype(jnp.int32))
    x2 = jnp.minimum(w - 1, (rois[..., 2] * w).astype(jnp.int32))
    y2 = jnp.minimum(h - 1, (rois[..., 3] * h).astype(jnp.int32))
    valid = (x2 >= x1) & (y2 >= y1)
    rw = x2 - x1 + 1
    rh = y2 - y1 + 1
    jj = jnp.arange(_OW)
    ii = jnp.arange(_OH)
    xs = x1[..., None] + (jj * rw[..., None]) // _OW
    xe = x1[..., None] + -((-(jj + 1) * rw[..., None]) // _OW)
    ys = y1[..., None] + (ii * rh[..., None]) // _OH
    ye = y1[..., None] + -((-(ii + 1) * rh[..., None]) // _OH)
    lenx = jnp.maximum(xe - xs, 1)
    kx = (lenx >= 2).astype(jnp.int32) + (lenx >= 4).astype(jnp.int32)
    xb = xe - jnp.left_shift(1, kx)
    xs_c = jnp.clip(xs, 0, w - 1)
    xb_c = jnp.clip(xb, 0, w - 1)
    bounds = jnp.concatenate(
        [xs_c, xb_c, kx, ys, ye, valid[..., None].astype(jnp.int32)], axis=-1
    )

    feat_t = features.transpose(0, 3, 2, 1).astype(jnp.bfloat16)

    out = pl.pallas_call(
        _roi_body,
        grid_spec=pltpu.PrefetchScalarGridSpec(
            num_scalar_prefetch=1,
            grid=(b, n),
            in_specs=[
                pl.BlockSpec((1, w, h, c), lambda pb, pn, bnds: (pb, 0, 0, 0)),
            ],
            out_specs=pl.BlockSpec(
                (1, 1, _OH, _OW * c), lambda pb, pn, bnds: (pb, pn, 0, 0)
            ),
            scratch_shapes=[pltpu.VMEM((3, w, h, c), jnp.bfloat16)],
        ),
        out_shape=jax.ShapeDtypeStruct((b, n, _OH, _OW * c), jnp.float32),
    )(bounds, feat_t)

    return out.reshape(b, n, _OH, _OW, c).transpose(0, 1, 4, 2, 3)


# speedup vs baseline: 1.9394x; 1.0276x over previous
import jax
import jax.numpy as jnp
from jax.experimental import pallas as pl
from jax.experimental.pallas import tpu as pltpu

_OH = 7
_OW = 7


def _roi_body(bounds_ref, feat_ref, out_ref, tx_ref):
    pb = pl.program_id(0)
    pn = pl.program_id(1)
    h = feat_ref.shape[2]

    @pl.when(pn == 0)
    def _build():
        t0 = feat_ref[0]
        t1 = jnp.maximum(t0, jnp.concatenate([t0[1:], t0[-1:]], axis=0))
        t2 = jnp.maximum(t1, jnp.concatenate([t1[2:], t1[-2:]], axis=0))
        tx_ref[0] = t0
        tx_ref[1] = t1
        tx_ref[2] = t2

    c = feat_ref.shape[3]
    neg = jnp.array(-jnp.inf, dtype=jnp.float32)
    zero = jnp.array(0.0, dtype=jnp.float32)
    ridx = jax.lax.broadcasted_iota(jnp.int32, (h, 1), 0)
    vflag = bounds_ref[pb, pn, 5 * _OW]
    rms = []
    for ii in range(_OH):
        ys = bounds_ref[pb, pn, 3 * _OW + ii]
        ye = bounds_ref[pb, pn, 4 * _OW + ii]
        rms.append((ridx >= ys) & (ridx < ye))
    for jj in range(_OW):
        xs = bounds_ref[pb, pn, jj]
        xb = bounds_ref[pb, pn, _OW + jj]
        kx = bounds_ref[pb, pn, 2 * _OW + jj]
        cm = jnp.maximum(tx_ref[kx, xs], tx_ref[kx, xb])
        for ii in range(_OH):
            cell = jnp.max(jnp.where(rms[ii], cm, neg), axis=0)
            out_ref[0, 0, ii, jj * c : (jj + 1) * c] = jnp.where(
                vflag > 0, cell, zero
            )


def kernel(features, rois):
    b, c, h, w = features.shape
    n = rois.shape[1]

    x1 = jnp.maximum(0, (rois[..., 0] * w).astype(jnp.int32))
    y1 = jnp.maximum(0, (rois[..., 1] * h).astype(jnp.int32))
    x2 = jnp.minimum(w - 1, (rois[..., 2] * w).astype(jnp.int32))
    y2 = jnp.minimum(h - 1, (rois[..., 3] * h).astype(jnp.int32))
    valid = (x2 >= x1) & (y2 >= y1)
    rw = x2 - x1 + 1
    rh = y2 - y1 + 1
    jj = jnp.arange(_OW)
    ii = jnp.arange(_OH)
    xs = x1[..., None] + (jj * rw[..., None]) // _OW
    xe = x1[..., None] + -((-(jj + 1) * rw[..., None]) // _OW)
    ys = y1[..., None] + (ii * rh[..., None]) // _OH
    ye = y1[..., None] + -((-(ii + 1) * rh[..., None]) // _OH)
    lenx = jnp.maximum(xe - xs, 1)
    kx = (lenx >= 2).astype(jnp.int32) + (lenx >= 4).astype(jnp.int32)
    xb = xe - jnp.left_shift(1, kx)
    xs_c = jnp.clip(xs, 0, w - 1)
    xb_c = jnp.clip(xb, 0, w - 1)
    bounds = jnp.concatenate(
        [xs_c, xb_c, kx, ys, ye, valid[..., None].astype(jnp.int32)], axis=-1
    )

    feat_t = features.transpose(0, 3, 2, 1)

    out = pl.pallas_call(
        _roi_body,
        grid_spec=pltpu.PrefetchScalarGridSpec(
            num_scalar_prefetch=1,
            grid=(b, n),
            in_specs=[
                pl.BlockSpec((1, w, h, c), lambda pb, pn, bnds: (pb, 0, 0, 0)),
            ],
            out_specs=pl.BlockSpec(
                (1, 1, _OH, _OW * c), lambda pb, pn, bnds: (pb, pn, 0, 0)
            ),
            scratch_shapes=[pltpu.VMEM((3, w, h, c), features.dtype)],
        ),
        out_shape=jax.ShapeDtypeStruct((b, n, _OH, _OW * c), features.dtype),
    )(bounds, feat_t)

    return out.reshape(b, n, _OH, _OW, c).transpose(0, 1, 4, 2, 3)


# speedup vs baseline: 1.9565x; 1.0089x over previous
import jax
import jax.numpy as jnp
from jax.experimental import pallas as pl
from jax.experimental.pallas import tpu as pltpu

_OH = 7
_OW = 7


def _roi_body(bounds_ref, feat_ref, out_ref, tx_ref):
    pb = pl.program_id(0)
    pn = pl.program_id(1)
    h = feat_ref.shape[2]

    @pl.when(pn == 0)
    def _build():
        t0 = feat_ref[0]
        t1 = jnp.maximum(t0, jnp.concatenate([t0[1:], t0[-1:]], axis=0))
        t2 = jnp.maximum(t1, jnp.concatenate([t1[2:], t1[-2:]], axis=0))
        tx_ref[0] = t0
        tx_ref[1] = t1
        tx_ref[2] = t2

    cms = []
    for jj in range(_OW):
        xs = bounds_ref[pb, pn, jj]
        xb = bounds_ref[pb, pn, _OW + jj]
        kx = bounds_ref[pb, pn, 2 * _OW + jj]
        cms.append(jnp.maximum(tx_ref[kx, xs], tx_ref[kx, xb]))
    cmall = jnp.concatenate(cms, axis=-1)

    neg = jnp.array(-jnp.inf, dtype=cmall.dtype)
    zero = jnp.array(0.0, dtype=cmall.dtype)
    ridx = jax.lax.broadcasted_iota(jnp.int32, (h, 1), 0)
    vflag = bounds_ref[pb, pn, 5 * _OW]
    for ii in range(_OH):
        ys = bounds_ref[pb, pn, 3 * _OW + ii]
        ye = bounds_ref[pb, pn, 4 * _OW + ii]
        rm = (ridx >= ys) & (ridx < ye)
        row = jnp.max(jnp.where(rm, cmall, neg), axis=0)
        out_ref[0, 0, ii, :] = jnp.where(vflag > 0, row, zero)


def kernel(features, rois):
    b, c, h, w = features.shape
    n = rois.shape[1]

    x1 = jnp.maximum(0, (rois[..., 0] * w).astype(jnp.int32))
    y1 = jnp.maximum(0, (rois[..., 1] * h).astype(jnp.int32))
    x2 = jnp.minimum(w - 1, (rois[..., 2] * w).astype(jnp.int32))
    y2 = jnp.minimum(h - 1, (rois[..., 3] * h).astype(jnp.int32))
    valid = (x2 >= x1) & (y2 >= y1)
    rw = x2 - x1 + 1
    rh = y2 - y1 + 1
    jj = jnp.arange(_OW)
    ii = jnp.arange(_OH)
    xs = x1[..., None] + (jj * rw[..., None]) // _OW
    xe = x1[..., None] + -((-(jj + 1) * rw[..., None]) // _OW)
    ys = y1[..., None] + (ii * rh[..., None]) // _OH
    ye = y1[..., None] + -((-(ii + 1) * rh[..., None]) // _OH)
    lenx = jnp.maximum(xe - xs, 1)
    kx = (lenx >= 2).astype(jnp.int32) + (lenx >= 4).astype(jnp.int32)
    xb = xe - jnp.left_shift(1, kx)
    xs_c = jnp.clip(xs, 0, w - 1)
    xb_c = jnp.clip(xb, 0, w - 1)
    bounds = jnp.concatenate(
        [xs_c, xb_c, kx, ys, ye, valid[..., None].astype(jnp.int32)], axis=-1
    )

    feat_t = features.transpose(0, 3, 2, 1)

    out = pl.pallas_call(
        _roi_body,
        grid_spec=pltpu.PrefetchScalarGridSpec(
            num_scalar_prefetch=1,
            grid=(b, n),
            in_specs=[
                pl.BlockSpec((1, w, h, c), lambda pb, pn, bnds: (pb, 0, 0, 0)),
            ],
            out_specs=pl.BlockSpec(
                (1, 1, _OH, _OW * c), lambda pb, pn, bnds: (pb, pn, 0, 0)
            ),
            scratch_shapes=[pltpu.VMEM((3, w, h, c), features.dtype)],
        ),
        out_shape=jax.ShapeDtypeStruct((b, n, _OH, _OW * c), features.dtype),
    )(bounds, feat_t)

    return out.reshape(b, n, _OH, _OW, c).transpose(0, 1, 4, 2, 3)
